# baseline (speedup 1.0000x reference)
import functools

import jax
import jax.numpy as jnp
from jax import lax
from jax.experimental import pallas as pl
from jax.experimental.pallas import tpu as pltpu
from jax.experimental.pallas import tpu_sc as plsc

_NC = 2
_NS = 16
_NW = _NC * _NS

_CB = 16
_L = 16


@functools.partial(jax.jit, static_argnums=(2, 3, 4))
def _embed(W, data, B, H, D):
    rows_per_w = B // _NW
    n_sup = rows_per_w // _CB
    mesh = plsc.VectorSubcoreMesh(
        core_axis_name="c", subcore_axis_name="s",
        num_cores=_NC, num_subcores=_NS,
    )

    @functools.partial(
        pl.kernel,
        out_type=jax.ShapeDtypeStruct((H, D, B), jnp.float32),
        mesh=mesh,
        scratch_types=[
            pltpu.VMEM((2, _CB, H), jnp.int32),
            pltpu.VMEM((2, _CB, H, D), jnp.float32),
            pltpu.VMEM((H, D, _CB), jnp.float32),
            pltpu.SemaphoreType.DMA,
        ],
        compiler_params=pltpu.CompilerParams(
            use_tc_tiling_on_sc=False, needs_layout_passes=False,
        ),
    )
    def k(table_hbm, idx_hbm, out_hbm, idx_v, rows_v, rows_t, gsem):
        wid = lax.axis_index("s") * _NC + lax.axis_index("c")
        base = wid * rows_per_w

        def load_idx(c, slot):
            row = pl.multiple_of(base + c * _CB, 8)
            pltpu.sync_copy(idx_hbm.at[pl.ds(row, _CB)], idx_v.at[slot])

        def fire(slot):
            for j in range(_CB):
                pltpu.async_copy(
                    table_hbm.at[idx_v.at[slot].at[j]],
                    rows_v.at[slot].at[j],
                    gsem,
                )

        def drain(slot):
            for j in range(_CB):
                pltpu.make_async_copy(
                    table_hbm.at[idx_v.at[slot].at[j]],
                    rows_v.at[slot].at[j],
                    gsem,
                ).wait()

        biota = lax.iota(jnp.int32, _L)

        def transpose_store(c, slot):
            src = rows_v.at[slot]

            def h_body(h, carry):
                hvec = jnp.full((_L,), h, dtype=jnp.int32)
                for bl in range(0, _CB, _L):
                    b_idx = biota + bl
                    for d in range(D):
                        dvec = jnp.full((_L,), d, dtype=jnp.int32)
                        v = plsc.load_gather(src, [b_idx, hvec, dvec])
                        rows_t[h, d, pl.ds(bl, _L)] = v
                return carry

            lax.fori_loop(0, H, h_body, 0)
            row = pl.multiple_of(base + c * _CB, 8)
            pltpu.sync_copy(rows_t, out_hbm.at[:, :, pl.ds(row, _CB)])

        load_idx(0, 0)
        fire(0)

        def body(p, carry):
            a = 2 * p
            b = a + 1

            load_idx(b, 1)
            fire(1)

            drain(0)
            transpose_store(a, 0)

            @pl.when(a + 2 < n_sup)
            def _():
                load_idx(a + 2, 0)
                fire(0)

            drain(1)
            transpose_store(b, 1)
            return carry

        lax.fori_loop(0, n_sup // 2, body, 0)

    return k(W, data)


def kernel(data, W):
    B, H = data.shape
    D = W.shape[1]
    out_t = _embed(W, data, B, H, D)
    return jnp.transpose(out_t, (2, 0, 1))


# speedup vs baseline: 1.3706x; 1.3706x over previous
import functools

import jax
import jax.numpy as jnp
from jax import lax
from jax.experimental import pallas as pl
from jax.experimental.pallas import tpu as pltpu
from jax.experimental.pallas import tpu_sc as plsc

_NC = 2
_NS = 16
_NW = _NC * _NS

_CB = 16


@functools.partial(jax.jit, static_argnums=(2, 3, 4))
def _embed(W, data, B, H, D):
    rows_per_w = B // _NW
    n_sup = rows_per_w // _CB
    mesh = plsc.VectorSubcoreMesh(
        core_axis_name="c", subcore_axis_name="s",
        num_cores=_NC, num_subcores=_NS,
    )

    @functools.partial(
        pl.kernel,
        out_type=jax.ShapeDtypeStruct((H, B, D), jnp.float32),
        mesh=mesh,
        scratch_types=[
            pltpu.VMEM((2, _CB, H), jnp.int32),
            pltpu.VMEM((2, _CB, H, D), jnp.float32),
            pltpu.SemaphoreType.DMA,
            pltpu.SemaphoreType.DMA,
            pltpu.SemaphoreType.DMA,
        ],
        compiler_params=pltpu.CompilerParams(
            use_tc_tiling_on_sc=False, needs_layout_passes=False,
        ),
    )
    def k(table_hbm, idx_hbm, out_hbm, idx_v, rows_v, gsem, osem0, osem1):
        wid = lax.axis_index("s") * _NC + lax.axis_index("c")
        base = wid * rows_per_w
        osem = (osem0, osem1)

        def load_idx(c, slot):
            row = pl.multiple_of(base + c * _CB, 8)
            pltpu.sync_copy(idx_hbm.at[pl.ds(row, _CB)], idx_v.at[slot])

        def fire(slot):
            for j in range(_CB):
                pltpu.async_copy(
                    table_hbm.at[idx_v.at[slot].at[j]],
                    rows_v.at[slot].at[j],
                    gsem,
                )

        def drain(slot):
            for j in range(_CB):
                pltpu.make_async_copy(
                    table_hbm.at[idx_v.at[slot].at[j]],
                    rows_v.at[slot].at[j],
                    gsem,
                ).wait()

        def out_fire(c, slot):
            row = pl.multiple_of(base + c * _CB, 8)
            for j in range(_CB):
                pltpu.async_copy(
                    rows_v.at[slot].at[j],
                    out_hbm.at[:, row + j, :],
                    osem[slot],
                )

        def out_drain(c, slot):
            row = pl.multiple_of(base + c * _CB, 8)
            for j in range(_CB):
                pltpu.make_async_copy(
                    rows_v.at[slot].at[j],
                    out_hbm.at[:, row + j, :],
                    osem[slot],
                ).wait()

        load_idx(0, 0)
        fire(0)

        def body(p, carry):
            a = 2 * p
            b = a + 1

            @pl.when(p > 0)
            def _():
                out_drain(b - 2, 1)

            load_idx(b, 1)
            fire(1)

            drain(0)
            out_fire(a, 0)

            drain(1)
            out_fire(b, 1)

            @pl.when(a + 2 < n_sup)
            def _():
                out_drain(a, 0)
                load_idx(a + 2, 0)
                fire(0)

            return carry

        lax.fori_loop(0, n_sup // 2, body, 0)
        out_drain(n_sup - 2, 0)
        out_drain(n_sup - 1, 1)

    return k(W, data)


def kernel(data, W):
    B, H = data.shape
    D = W.shape[1]
    out_t = _embed(W, data, B, H, D)
    return jnp.transpose(out_t, (1, 0, 2))


# speedup vs baseline: 1.3714x; 1.0006x over previous
import functools

import jax
import jax.numpy as jnp
from jax import lax
from jax.experimental import pallas as pl
from jax.experimental.pallas import tpu as pltpu
from jax.experimental.pallas import tpu_sc as plsc

_NC = 2
_NS = 16
_NW = _NC * _NS

_CB = 32


@functools.partial(jax.jit, static_argnums=(2, 3, 4))
def _embed(W, data, B, H, D):
    rows_per_w = B // _NW
    n_sup = rows_per_w // _CB
    mesh = plsc.VectorSubcoreMesh(
        core_axis_name="c", subcore_axis_name="s",
        num_cores=_NC, num_subcores=_NS,
    )

    @functools.partial(
        pl.kernel,
        out_type=jax.ShapeDtypeStruct((H, B, D), jnp.float32),
        mesh=mesh,
        scratch_types=[
            pltpu.VMEM((2, _CB, H), jnp.int32),
            pltpu.VMEM((2, _CB, H, D), jnp.float32),
            pltpu.SemaphoreType.DMA,
            pltpu.SemaphoreType.DMA,
            pltpu.SemaphoreType.DMA,
        ],
        compiler_params=pltpu.CompilerParams(
            use_tc_tiling_on_sc=False, needs_layout_passes=False,
        ),
    )
    def k(table_hbm, idx_hbm, out_hbm, idx_v, rows_v, gsem, osem0, osem1):
        wid = lax.axis_index("s") * _NC + lax.axis_index("c")
        base = wid * rows_per_w
        osem = (osem0, osem1)

        def load_idx(c, slot):
            row = pl.multiple_of(base + c * _CB, 8)
            pltpu.sync_copy(idx_hbm.at[pl.ds(row, _CB)], idx_v.at[slot])

        def fire(slot):
            for j in range(_CB):
                pltpu.async_copy(
                    table_hbm.at[idx_v.at[slot].at[j]],
                    rows_v.at[slot].at[j],
                    gsem,
                )

        def drain(slot):
            for j in range(_CB):
                pltpu.make_async_copy(
                    table_hbm.at[idx_v.at[slot].at[j]],
                    rows_v.at[slot].at[j],
                    gsem,
                ).wait()

        def out_fire(c, slot):
            row = pl.multiple_of(base + c * _CB, 8)
            for j in range(_CB):
                pltpu.async_copy(
                    rows_v.at[slot].at[j],
                    out_hbm.at[:, row + j, :],
                    osem[slot],
                )

        def out_drain(c, slot):
            row = pl.multiple_of(base + c * _CB, 8)
            for j in range(_CB):
                pltpu.make_async_copy(
                    rows_v.at[slot].at[j],
                    out_hbm.at[:, row + j, :],
                    osem[slot],
                ).wait()

        load_idx(0, 0)
        fire(0)

        def body(p, carry):
            a = 2 * p
            b = a + 1

            @pl.when(p > 0)
            def _():
                out_drain(b - 2, 1)

            load_idx(b, 1)
            fire(1)

            drain(0)
            out_fire(a, 0)

            drain(1)
            out_fire(b, 1)

            @pl.when(a + 2 < n_sup)
            def _():
                out_drain(a, 0)
                load_idx(a + 2, 0)
                fire(0)

            return carry

        lax.fori_loop(0, n_sup // 2, body, 0)
        out_drain(n_sup - 2, 0)
        out_drain(n_sup - 1, 1)

    return k(W, data)


def kernel(data, W):
    B, H = data.shape
    D = W.shape[1]
    out_t = _embed(W, data, B, H, D)
    return jnp.transpose(out_t, (1, 0, 2))


# speedup vs baseline: 1.4918x; 1.0878x over previous
import functools

import jax
import jax.numpy as jnp
from jax import lax
from jax.experimental import pallas as pl
from jax.experimental.pallas import tpu as pltpu
from jax.experimental.pallas import tpu_sc as plsc

_NC = 2
_NS = 16
_NW = _NC * _NS

_CB = 16
_L = 16


@functools.partial(jax.jit, static_argnums=(2, 3, 4))
def _embed(W, data, B, H, D):
    rows_per_w = B // _NW
    n_sup = rows_per_w // _CB
    mesh = plsc.VectorSubcoreMesh(
        core_axis_name="c", subcore_axis_name="s",
        num_cores=_NC, num_subcores=_NS,
    )

    @functools.partial(
        pl.kernel,
        out_type=jax.ShapeDtypeStruct((H * D, B), jnp.float32),
        mesh=mesh,
        scratch_types=[
            pltpu.VMEM((2, _CB, H), jnp.int32),
            pltpu.VMEM((2, _CB * H, D), jnp.float32),
            pltpu.VMEM((H * D, _CB), jnp.float32),
            pltpu.SemaphoreType.DMA,
        ],
        compiler_params=pltpu.CompilerParams(
            use_tc_tiling_on_sc=False, needs_layout_passes=False,
        ),
    )
    def k(table_hbm, idx_hbm, out_hbm, idx_v, rows_v, rows_t, gsem):
        wid = lax.axis_index("s") * _NC + lax.axis_index("c")
        base = wid * rows_per_w

        def load_idx(c, slot):
            row = pl.multiple_of(base + c * _CB, 8)
            pltpu.sync_copy(idx_hbm.at[pl.ds(row, _CB)], idx_v.at[slot])

        def fire(slot):
            for j in range(_CB):
                pltpu.async_copy(
                    table_hbm.at[idx_v.at[slot].at[j]],
                    rows_v.at[slot].at[pl.ds(j * H, H)],
                    gsem,
                )

        def drain(slot):
            for j in range(_CB):
                pltpu.make_async_copy(
                    table_hbm.at[idx_v.at[slot].at[j]],
                    rows_v.at[slot].at[pl.ds(j * H, H)],
                    gsem,
                ).wait()

        diota = lax.iota(jnp.int32, _L)
        bvecs = [jnp.full((_L,), b, dtype=jnp.int32) for b in range(_CB)]

        def transpose_store(c, slot):
            src = rows_v.at[slot]

            def h_body(h, carry):
                for d0 in range(0, D, _L):
                    r_idx = diota + (h * D + d0)
                    for b in range(_CB):
                        v = src[b * H + h, pl.ds(d0, _L)]
                        plsc.store_scatter(rows_t, [r_idx, bvecs[b]], v)
                return carry

            lax.fori_loop(0, H, h_body, 0)
            row = pl.multiple_of(base + c * _CB, 8)
            pltpu.sync_copy(rows_t, out_hbm.at[:, pl.ds(row, _CB)])

        load_idx(0, 0)
        fire(0)

        def body(p, carry):
            a = 2 * p
            b = a + 1

            load_idx(b, 1)
            fire(1)

            drain(0)
            transpose_store(a, 0)

            @pl.when(a + 2 < n_sup)
            def _():
                load_idx(a + 2, 0)
                fire(0)

            drain(1)
            transpose_store(b, 1)
            return carry

        lax.fori_loop(0, n_sup // 2, body, 0)

    return k(W, data)


def kernel(data, W):
    B, H = data.shape
    D = W.shape[1]
    out2 = _embed(W, data, B, H, D)
    return jnp.transpose(out2.reshape(H, D, B), (2, 0, 1))


# speedup vs baseline: 1.5337x; 1.0281x over previous
import functools

import jax
import jax.numpy as jnp
from jax import lax
from jax.experimental import pallas as pl
from jax.experimental.pallas import tpu as pltpu
from jax.experimental.pallas import tpu_sc as plsc

_NC = 2
_NS = 16
_NW = _NC * _NS

_CB = 16
_L = 16


@functools.partial(jax.jit, static_argnums=(2, 3, 4))
def _embed(W, data, B, H, D):
    rows_per_w = B // _NW
    n_sup = rows_per_w // _CB
    mesh = plsc.VectorSubcoreMesh(
        core_axis_name="c", subcore_axis_name="s",
        num_cores=_NC, num_subcores=_NS,
    )

    @functools.partial(
        pl.kernel,
        out_type=jax.ShapeDtypeStruct((H * D, B), jnp.float32),
        mesh=mesh,
        scratch_types=[
            pltpu.VMEM((2, _CB, H), jnp.int32),
            pltpu.VMEM((2, _CB * H, D), jnp.float32),
            pltpu.VMEM((H * D, _CB), jnp.float32),
            pltpu.SemaphoreType.DMA,
            pltpu.SemaphoreType.DMA,
        ],
        compiler_params=pltpu.CompilerParams(
            use_tc_tiling_on_sc=False, needs_layout_passes=False,
        ),
    )
    def k(table_hbm, idx_hbm, out_hbm, idx_v, rows_v, rows_t, gsem, osem):
        wid = lax.axis_index("s") * _NC + lax.axis_index("c")
        base = wid * rows_per_w

        def load_idx(c, slot):
            row = pl.multiple_of(base + c * _CB, 8)
            pltpu.sync_copy(idx_hbm.at[pl.ds(row, _CB)], idx_v.at[slot])

        def fire(slot):
            for j in range(_CB):
                pltpu.async_copy(
                    table_hbm.at[idx_v.at[slot].at[j]],
                    rows_v.at[slot].at[pl.ds(j * H, H)],
                    gsem,
                )

        def drain(slot):
            for j in range(_CB):
                pltpu.make_async_copy(
                    table_hbm.at[idx_v.at[slot].at[j]],
                    rows_v.at[slot].at[pl.ds(j * H, H)],
                    gsem,
                ).wait()

        diota = lax.iota(jnp.int32, _L)
        bvecs = [jnp.full((_L,), b, dtype=jnp.int32) for b in range(_CB)]

        def out_copy(c):
            row = pl.multiple_of(base + c * _CB, 8)
            return pltpu.make_async_copy(
                rows_t, out_hbm.at[:, pl.ds(row, _CB)], osem,
            )

        def transpose_store(c, slot):
            src = rows_v.at[slot]

            @pl.when(c > 0)
            def _():
                out_copy(c).wait()

            def h_body(h, carry):
                for d0 in range(0, D, _L):
                    r_idx = diota + (h * D + d0)
                    for b in range(_CB):
                        v = src[b * H + h, pl.ds(d0, _L)]
                        plsc.store_scatter(rows_t, [r_idx, bvecs[b]], v)
                return carry

            lax.fori_loop(0, H, h_body, 0)
            out_copy(c).start()

        load_idx(0, 0)
        fire(0)

        def body(p, carry):
            a = 2 * p
            b = a + 1

            load_idx(b, 1)
            fire(1)

            drain(0)
            transpose_store(a, 0)

            @pl.when(a + 2 < n_sup)
            def _():
                load_idx(a + 2, 0)
                fire(0)

            drain(1)
            transpose_store(b, 1)
            return carry

        lax.fori_loop(0, n_sup // 2, body, 0)
        out_copy(n_sup - 1).wait()

    return k(W, data)


def kernel(data, W):
    B, H = data.shape
    D = W.shape[1]
    out2 = _embed(W, data, B, H, D)
    return jnp.transpose(out2.reshape(H, D, B), (2, 0, 1))
